# Initial kernel scaffold; baseline (speedup 1.0000x reference)
import jax
import jax.numpy as jnp
from jax.experimental import pallas as pl


def kernel(distribution, boxes, features, labels, obj_embed_w, bn4_gamma, bn4_beta, pos_W, pos_b, W1, b1, bn2_gamma, bn2_beta, W2, b2):
    raise NotImplementedError("write your pallas kernel here")


# speedup vs baseline: 1.5456x; 1.5456x over previous
import jax
import jax.numpy as jnp
from jax.experimental import pallas as pl
from jax.experimental.pallas import tpu as pltpu

N = 20000
R = 1000
EPS = 1e-5


def _center_size_block(boxes_blk):
    b = boxes_blk[:, 1:5]
    wh = b[:, 2:4] - b[:, 0:2] + 1.0
    ctr = b[:, 0:2] + 0.5 * wh
    return jnp.concatenate([ctr, wh], axis=1)


def _stats_kernel(boxes_ref, mu_ref, var_ref):
    cs = _center_size_block(boxes_ref[...])
    mu = jnp.mean(cs, axis=0, keepdims=True)
    var = jnp.mean(cs * cs, axis=0, keepdims=True) - mu * mu
    mu_ref[...] = mu
    var_ref[...] = var


def _main_kernel(boxes_ref, dist_ref, feat_ref, a4t_ref, beff_ref,
                 wdt_ref, wpt_ref, wft_ref, b1_ref,
                 z_ref, s_ref, ss_ref):
    i = pl.program_id(0)
    cs = _center_size_block(boxes_ref[...])
    pe = jnp.maximum(
        jnp.dot(cs, a4t_ref[...], preferred_element_type=jnp.float32)
        + beff_ref[...], 0.0)
    z = (jnp.dot(feat_ref[...], wft_ref[...], preferred_element_type=jnp.float32)
         + jnp.dot(dist_ref[...], wdt_ref[...], preferred_element_type=jnp.float32)
         + jnp.dot(pe, wpt_ref[...], preferred_element_type=jnp.float32)
         + b1_ref[...])
    z_ref[...] = z
    zs = jnp.sum(z, axis=0, keepdims=True)
    zss = jnp.sum(z * z, axis=0, keepdims=True)

    @pl.when(i == 0)
    def _init():
        s_ref[...] = zs
        ss_ref[...] = zss

    @pl.when(i > 0)
    def _acc():
        s_ref[...] += zs
        ss_ref[...] += zss


def _finish_kernel(z_ref, scale_ref, shift_ref, w2t_ref, b2_ref, out_ref):
    h = jnp.maximum(z_ref[...] * scale_ref[...] + shift_ref[...], 0.0)
    out_ref[...] = (jnp.dot(h, w2t_ref[...], preferred_element_type=jnp.float32)
                    + b2_ref[...])


def kernel(distribution, boxes, features, labels, obj_embed_w, bn4_gamma,
           bn4_beta, pos_W, pos_b, W1, b1, bn2_gamma, bn2_beta, W2, b2):
    nb = N // R

    mu4, var4 = pl.pallas_call(
        _stats_kernel,
        out_shape=(jax.ShapeDtypeStruct((1, 4), jnp.float32),
                   jax.ShapeDtypeStruct((1, 4), jnp.float32)),
    )(boxes)
    mu4 = mu4[0]
    var4 = var4[0]

    scale4 = bn4_gamma * jax.lax.rsqrt(var4 + EPS)
    shift4 = bn4_beta - mu4 * scale4
    a4t = (pos_W * scale4[None, :]).T
    beff = (pos_b + pos_W @ shift4)[None, :]
    wft = W1[:, :2048].T
    wdt = obj_embed_w @ W1[:, 2048:2248].T
    wpt = W1[:, 2248:2376].T
    b1r = b1[None, :]

    z, s, ss = pl.pallas_call(
        _main_kernel,
        grid=(nb,),
        in_specs=[
            pl.BlockSpec((R, 5), lambda i: (i, 0)),
            pl.BlockSpec((R, 36), lambda i: (i, 0)),
            pl.BlockSpec((R, 2048), lambda i: (i, 0)),
            pl.BlockSpec((4, 128), lambda i: (0, 0)),
            pl.BlockSpec((1, 128), lambda i: (0, 0)),
            pl.BlockSpec((36, 1024), lambda i: (0, 0)),
            pl.BlockSpec((128, 1024), lambda i: (0, 0)),
            pl.BlockSpec((2048, 1024), lambda i: (0, 0)),
            pl.BlockSpec((1, 1024), lambda i: (0, 0)),
        ],
        out_specs=(
            pl.BlockSpec((R, 1024), lambda i: (i, 0)),
            pl.BlockSpec((1, 1024), lambda i: (0, 0)),
            pl.BlockSpec((1, 1024), lambda i: (0, 0)),
        ),
        out_shape=(
            jax.ShapeDtypeStruct((N, 1024), jnp.float32),
            jax.ShapeDtypeStruct((1, 1024), jnp.float32),
            jax.ShapeDtypeStruct((1, 1024), jnp.float32),
        ),
        compiler_params=pltpu.CompilerParams(
            dimension_semantics=("arbitrary",)),
    )(boxes, distribution, features, a4t, beff, wdt, wpt, wft, b1r)

    mu_z = s / N
    var_z = ss / N - mu_z * mu_z
    scale = bn2_gamma[None, :] * jax.lax.rsqrt(var_z + EPS)
    shift = bn2_beta[None, :] - mu_z * scale
    w2t = W2.T
    b2r = b2[None, :]

    dist_out = pl.pallas_call(
        _finish_kernel,
        grid=(nb,),
        in_specs=[
            pl.BlockSpec((R, 1024), lambda i: (i, 0)),
            pl.BlockSpec((1, 1024), lambda i: (0, 0)),
            pl.BlockSpec((1, 1024), lambda i: (0, 0)),
            pl.BlockSpec((1024, 37), lambda i: (0, 0)),
            pl.BlockSpec((1, 37), lambda i: (0, 0)),
        ],
        out_specs=pl.BlockSpec((R, 37), lambda i: (i, 0)),
        out_shape=jax.ShapeDtypeStruct((N, 37), jnp.float32),
        compiler_params=pltpu.CompilerParams(
            dimension_semantics=("arbitrary",)),
    )(z, scale, shift, w2t, b2r)

    return (dist_out, labels)


# speedup vs baseline: 1.5880x; 1.0275x over previous
import jax
import jax.numpy as jnp
from jax.experimental import pallas as pl
from jax.experimental.pallas import tpu as pltpu

N = 20000
R = 1000
EPS = 1e-5


def _center_size_block(boxes_blk):
    b = boxes_blk[:, 1:5]
    wh = b[:, 2:4] - b[:, 0:2] + 1.0
    ctr = b[:, 0:2] + 0.5 * wh
    return jnp.concatenate([ctr, wh], axis=1)


def _stats_kernel(boxes_ref, mu_ref, var_ref):
    cs = _center_size_block(boxes_ref[...])
    mu = jnp.mean(cs, axis=0, keepdims=True)
    var = jnp.mean(cs * cs, axis=0, keepdims=True) - mu * mu
    mu_ref[...] = mu
    var_ref[...] = var


def _main_kernel(boxes_ref, dist_ref, feat_ref, a4t_ref, beff_ref,
                 wdt_ref, wpt_ref, wft_ref, b1_ref,
                 z_ref, s_ref, ss_ref):
    i = pl.program_id(0)
    cs = _center_size_block(boxes_ref[...])
    pe = jnp.maximum(
        jnp.dot(cs, a4t_ref[...], preferred_element_type=jnp.float32)
        + beff_ref[...], 0.0)
    z = (jnp.dot(feat_ref[...], wft_ref[...], preferred_element_type=jnp.float32)
         + jnp.dot(dist_ref[...], wdt_ref[...], preferred_element_type=jnp.float32)
         + jnp.dot(pe, wpt_ref[...], preferred_element_type=jnp.float32)
         + b1_ref[...])
    z_ref[...] = z.astype(z_ref.dtype)
    zs = jnp.sum(z, axis=0, keepdims=True)
    zss = jnp.sum(z * z, axis=0, keepdims=True)

    @pl.when(i == 0)
    def _init():
        s_ref[...] = zs
        ss_ref[...] = zss

    @pl.when(i > 0)
    def _acc():
        s_ref[...] += zs
        ss_ref[...] += zss


def _finish_kernel(z_ref, scale_ref, shift_ref, w2t_ref, b2_ref, out_ref):
    zf = z_ref[...].astype(jnp.float32)
    h = jnp.maximum(zf * scale_ref[...] + shift_ref[...], 0.0)
    out_ref[...] = (jnp.dot(h, w2t_ref[...], preferred_element_type=jnp.float32)
                    + b2_ref[...])


def kernel(distribution, boxes, features, labels, obj_embed_w, bn4_gamma,
           bn4_beta, pos_W, pos_b, W1, b1, bn2_gamma, bn2_beta, W2, b2):
    nb = N // R

    mu4, var4 = pl.pallas_call(
        _stats_kernel,
        out_shape=(jax.ShapeDtypeStruct((1, 4), jnp.float32),
                   jax.ShapeDtypeStruct((1, 4), jnp.float32)),
    )(boxes)
    mu4 = mu4[0]
    var4 = var4[0]

    scale4 = bn4_gamma * jax.lax.rsqrt(var4 + EPS)
    shift4 = bn4_beta - mu4 * scale4
    a4t = (pos_W * scale4[None, :]).T
    beff = (pos_b + pos_W @ shift4)[None, :]
    wft = W1[:, :2048].T
    wdt = obj_embed_w @ W1[:, 2048:2248].T
    wpt = W1[:, 2248:2376].T
    b1r = b1[None, :]

    z, s, ss = pl.pallas_call(
        _main_kernel,
        grid=(nb,),
        in_specs=[
            pl.BlockSpec((R, 5), lambda i: (i, 0)),
            pl.BlockSpec((R, 36), lambda i: (i, 0)),
            pl.BlockSpec((R, 2048), lambda i: (i, 0)),
            pl.BlockSpec((4, 128), lambda i: (0, 0)),
            pl.BlockSpec((1, 128), lambda i: (0, 0)),
            pl.BlockSpec((36, 1024), lambda i: (0, 0)),
            pl.BlockSpec((128, 1024), lambda i: (0, 0)),
            pl.BlockSpec((2048, 1024), lambda i: (0, 0)),
            pl.BlockSpec((1, 1024), lambda i: (0, 0)),
        ],
        out_specs=(
            pl.BlockSpec((R, 1024), lambda i: (i, 0)),
            pl.BlockSpec((1, 1024), lambda i: (0, 0)),
            pl.BlockSpec((1, 1024), lambda i: (0, 0)),
        ),
        out_shape=(
            jax.ShapeDtypeStruct((N, 1024), jnp.bfloat16),
            jax.ShapeDtypeStruct((1, 1024), jnp.float32),
            jax.ShapeDtypeStruct((1, 1024), jnp.float32),
        ),
        compiler_params=pltpu.CompilerParams(
            dimension_semantics=("arbitrary",)),
    )(boxes, distribution, features, a4t, beff, wdt, wpt, wft, b1r)

    mu_z = s / N
    var_z = ss / N - mu_z * mu_z
    scale = bn2_gamma[None, :] * jax.lax.rsqrt(var_z + EPS)
    shift = bn2_beta[None, :] - mu_z * scale
    w2t = W2.T
    b2r = b2[None, :]

    dist_out = pl.pallas_call(
        _finish_kernel,
        grid=(nb,),
        in_specs=[
            pl.BlockSpec((R, 1024), lambda i: (i, 0)),
            pl.BlockSpec((1, 1024), lambda i: (0, 0)),
            pl.BlockSpec((1, 1024), lambda i: (0, 0)),
            pl.BlockSpec((1024, 37), lambda i: (0, 0)),
            pl.BlockSpec((1, 37), lambda i: (0, 0)),
        ],
        out_specs=pl.BlockSpec((R, 37), lambda i: (i, 0)),
        out_shape=jax.ShapeDtypeStruct((N, 37), jnp.float32),
        compiler_params=pltpu.CompilerParams(
            dimension_semantics=("arbitrary",)),
    )(z, scale, shift, w2t, b2r)

    return (dist_out, labels)


# speedup vs baseline: 1.7257x; 1.0867x over previous
import jax
import jax.numpy as jnp
from jax.experimental import pallas as pl
from jax.experimental.pallas import tpu as pltpu

N = 20000
R = 1000
EPS = 1e-5


def _stats_kernel(bp_ref, out_ref):
    x = bp_ref[...]
    r = jnp.roll(x, -2, axis=1)
    wh = r - x + 1.0
    ctr = x + 0.5 * wh
    out_ref[...] = jnp.stack([
        jnp.sum(ctr, axis=0),
        jnp.sum(wh, axis=0),
        jnp.sum(ctr * ctr, axis=0),
        jnp.sum(wh * wh, axis=0),
    ])


def _main_kernel(boxes_ref, dist_ref, feat_ref, b5_ref, c_ref,
                 wdt_ref, wpt_ref, wft_ref, b1_ref,
                 z_ref, s_ref, ss_ref):
    i = pl.program_id(0)
    pe = jnp.maximum(
        jnp.dot(boxes_ref[...], b5_ref[...], preferred_element_type=jnp.float32)
        + c_ref[...], 0.0)
    z = (jnp.dot(feat_ref[...].astype(jnp.bfloat16), wft_ref[...],
                 preferred_element_type=jnp.float32)
         + jnp.dot(dist_ref[...], wdt_ref[...], preferred_element_type=jnp.float32)
         + jnp.dot(pe, wpt_ref[...], preferred_element_type=jnp.float32)
         + b1_ref[...])
    z_ref[...] = z.astype(z_ref.dtype)
    zs = jnp.sum(z, axis=0, keepdims=True)
    zss = jnp.sum(z * z, axis=0, keepdims=True)

    @pl.when(i == 0)
    def _init():
        s_ref[...] = zs
        ss_ref[...] = zss

    @pl.when(i > 0)
    def _acc():
        s_ref[...] += zs
        ss_ref[...] += zss


def _finish_kernel(z_ref, scale_ref, shift_ref, w2t_ref, b2_ref, out_ref):
    zf = z_ref[...].astype(jnp.float32)
    h = jnp.maximum(zf * scale_ref[...] + shift_ref[...], 0.0)
    out_ref[...] = (jnp.dot(h, w2t_ref[...], preferred_element_type=jnp.float32)
                    + b2_ref[...])


def kernel(distribution, boxes, features, labels, obj_embed_w, bn4_gamma,
           bn4_beta, pos_W, pos_b, W1, b1, bn2_gamma, bn2_beta, W2, b2):
    nb = N // R

    stats = pl.pallas_call(
        _stats_kernel,
        out_shape=jax.ShapeDtypeStruct((4, 160), jnp.float32),
    )(boxes.reshape(625, 160))
    st = stats.reshape(4, 32, 5).sum(axis=1)
    sum_ctr, sum_wh, sum_ctr2, sum_wh2 = st[0], st[1], st[2], st[3]
    mu4 = jnp.concatenate([sum_ctr[1:3], sum_wh[1:3]]) / N
    ex2 = jnp.concatenate([sum_ctr2[1:3], sum_wh2[1:3]]) / N
    var4 = ex2 - mu4 * mu4

    scale4 = bn4_gamma * jax.lax.rsqrt(var4 + EPS)
    shift4 = bn4_beta - mu4 * scale4
    a4t = (pos_W * scale4[None, :]).T
    beff = (pos_b + pos_W @ shift4)[None, :]
    b5 = jnp.stack([
        jnp.zeros((128,), jnp.float32),
        0.5 * a4t[0] - a4t[2],
        0.5 * a4t[1] - a4t[3],
        0.5 * a4t[0] + a4t[2],
        0.5 * a4t[1] + a4t[3],
    ])
    c = beff + 0.5 * a4t[0] + 0.5 * a4t[1] + a4t[2] + a4t[3]
    wft = W1[:, :2048].T.astype(jnp.bfloat16)
    wdt = obj_embed_w @ W1[:, 2048:2248].T
    wpt = W1[:, 2248:2376].T
    b1r = b1[None, :]

    z, s, ss = pl.pallas_call(
        _main_kernel,
        grid=(nb,),
        in_specs=[
            pl.BlockSpec((R, 5), lambda i: (i, 0)),
            pl.BlockSpec((R, 36), lambda i: (i, 0)),
            pl.BlockSpec((R, 2048), lambda i: (i, 0)),
            pl.BlockSpec((5, 128), lambda i: (0, 0)),
            pl.BlockSpec((1, 128), lambda i: (0, 0)),
            pl.BlockSpec((36, 1024), lambda i: (0, 0)),
            pl.BlockSpec((128, 1024), lambda i: (0, 0)),
            pl.BlockSpec((2048, 1024), lambda i: (0, 0)),
            pl.BlockSpec((1, 1024), lambda i: (0, 0)),
        ],
        out_specs=(
            pl.BlockSpec((R, 1024), lambda i: (i, 0)),
            pl.BlockSpec((1, 1024), lambda i: (0, 0)),
            pl.BlockSpec((1, 1024), lambda i: (0, 0)),
        ),
        out_shape=(
            jax.ShapeDtypeStruct((N, 1024), jnp.bfloat16),
            jax.ShapeDtypeStruct((1, 1024), jnp.float32),
            jax.ShapeDtypeStruct((1, 1024), jnp.float32),
        ),
        compiler_params=pltpu.CompilerParams(
            dimension_semantics=("arbitrary",)),
    )(boxes, distribution, features, b5, c, wdt, wpt, wft, b1r)

    mu_z = s / N
    var_z = ss / N - mu_z * mu_z
    scale = bn2_gamma[None, :] * jax.lax.rsqrt(var_z + EPS)
    shift = bn2_beta[None, :] - mu_z * scale
    w2t = W2.T
    b2r = b2[None, :]

    dist_out = pl.pallas_call(
        _finish_kernel,
        grid=(nb,),
        in_specs=[
            pl.BlockSpec((R, 1024), lambda i: (i, 0)),
            pl.BlockSpec((1, 1024), lambda i: (0, 0)),
            pl.BlockSpec((1, 1024), lambda i: (0, 0)),
            pl.BlockSpec((1024, 37), lambda i: (0, 0)),
            pl.BlockSpec((1, 37), lambda i: (0, 0)),
        ],
        out_specs=pl.BlockSpec((R, 37), lambda i: (i, 0)),
        out_shape=jax.ShapeDtypeStruct((N, 37), jnp.float32),
        compiler_params=pltpu.CompilerParams(
            dimension_semantics=("arbitrary",)),
    )(z, scale, shift, w2t, b2r)

    return (dist_out, labels)
